# baseline (speedup 1.0000x reference)
import functools

import jax
import jax.numpy as jnp
from jax import lax
from jax.experimental import pallas as pl
from jax.experimental.pallas import tpu as pltpu
from jax.experimental.pallas import tpu_sc as plsc

NC = 2
NS = 16
NW = NC * NS


def _make_sc_agg(n_nodes, d, n_edges, CH, NBUF, tc_tiling):
  epw = n_edges // NW
  n_chunks = epw // CH
  assert epw % CH == 0 and n_edges % NW == 0 and n_chunks % NBUF == 0
  rows_per_tile = (n_nodes + NS * 8 - 1) // (NS * 8) * 8
  n_pad = rows_per_tile * NS

  mesh = plsc.VectorSubcoreMesh(
      core_axis_name="c", subcore_axis_name="s", num_cores=NC,
      num_subcores=NS)

  @functools.partial(
      pl.kernel,
      out_type=jax.ShapeDtypeStruct((NC, n_pad, d), jnp.float32),
      mesh=mesh,
      compiler_params=pltpu.CompilerParams(use_tc_tiling_on_sc=tc_tiling),
      scratch_types=[
          pltpu.VMEM((n_chunks, CH), jnp.int32),
          pltpu.VMEM((n_chunks, CH), jnp.int32),
          *[pltpu.VMEM((CH, d), jnp.float32) for _ in range(NBUF)],
          pltpu.VMEM_SHARED((n_pad, d), jnp.float32),
          *[pltpu.SemaphoreType.DMA for _ in range(2 * NBUF)],
      ],
  )
  def sc_agg(x_hbm, src_hbm, dst_hbm, zeros_hbm, out_hbm,
             sidx, didx, *rest):
    rows = rest[:NBUF]
    agg_sh = rest[NBUF]
    semg = rest[NBUF + 1:2 * NBUF + 1]
    sems = rest[2 * NBUF + 1:]
    c = lax.axis_index("c")
    s = lax.axis_index("s")
    wid = s * NC + c
    r0 = s * rows_per_tile

    pltpu.sync_copy(src_hbm.at[wid], sidx)
    pltpu.sync_copy(dst_hbm.at[wid], didx)
    pltpu.sync_copy(zeros_hbm, agg_sh.at[pl.ds(r0, rows_per_tile)])
    plsc.subcore_barrier()

    for b in range(NBUF - 1):
      pltpu.async_copy(x_hbm.at[sidx.at[b]], rows[b], semg[b])

    def outer(t, carry):
      for b in range(NBUF):
        j = t * NBUF + b
        bg = (b + NBUF - 1) % NBUF
        jj = j + NBUF - 1

        @pl.when(jj < n_chunks)
        def _():
          @pl.when(j >= 1)
          def _():
            pltpu.make_async_copy(
                rows[bg], agg_sh.at[didx.at[0]], sems[bg]).wait()
          pltpu.async_copy(x_hbm.at[sidx.at[jj]], rows[bg], semg[bg])

        pltpu.make_async_copy(
            x_hbm.at[pl.ds(0, CH)], rows[b], semg[b]).wait()
        pltpu.async_copy(rows[b], agg_sh.at[didx.at[j]], sems[b], add=True)
      return carry

    lax.fori_loop(0, n_chunks // NBUF, outer, 0)
    for b in range(NBUF):
      pltpu.make_async_copy(rows[b], agg_sh.at[didx.at[0]], sems[b]).wait()
    plsc.subcore_barrier()

    pltpu.sync_copy(agg_sh.at[pl.ds(r0, rows_per_tile)],
                    out_hbm.at[c, pl.ds(r0, rows_per_tile)])

  return sc_agg


def _make_sc_agg_ring(n_nodes, d, n_edges, CH, NBUF, tc_tiling):
  epw = n_edges // NW
  n_chunks = epw // CH
  assert epw % CH == 0 and n_edges % NW == 0 and n_chunks >= NBUF >= 3
  rows_per_tile = (n_nodes + NS * 8 - 1) // (NS * 8) * 8
  n_pad = rows_per_tile * NS

  mesh = plsc.VectorSubcoreMesh(
      core_axis_name="c", subcore_axis_name="s", num_cores=NC,
      num_subcores=NS)

  @functools.partial(
      pl.kernel,
      out_type=jax.ShapeDtypeStruct((NC, n_pad, d), jnp.float32),
      mesh=mesh,
      compiler_params=pltpu.CompilerParams(use_tc_tiling_on_sc=tc_tiling),
      scratch_types=[
          *[pltpu.VMEM((CH,), jnp.int32) for _ in range(NBUF)],
          *[pltpu.VMEM((CH,), jnp.int32) for _ in range(NBUF)],
          *[pltpu.VMEM((CH, d), jnp.float32) for _ in range(NBUF)],
          pltpu.VMEM_SHARED((n_pad, d), jnp.float32),
          *[pltpu.SemaphoreType.DMA for _ in range(4 * NBUF)],
      ],
  )
  def sc_agg(x_hbm, src_hbm, dst_hbm, zeros_hbm, out_hbm, *rest):
    sbuf = rest[:NBUF]
    dbuf = rest[NBUF:2 * NBUF]
    rows = rest[2 * NBUF:3 * NBUF]
    agg_sh = rest[3 * NBUF]
    semsi = rest[3 * NBUF + 1:4 * NBUF + 1]
    semdi = rest[4 * NBUF + 1:5 * NBUF + 1]
    semg = rest[5 * NBUF + 1:6 * NBUF + 1]
    sems = rest[6 * NBUF + 1:]
    c = lax.axis_index("c")
    s = lax.axis_index("s")
    wid = s * NC + c
    r0 = s * rows_per_tile

    pltpu.sync_copy(zeros_hbm, agg_sh.at[pl.ds(r0, rows_per_tile)])
    plsc.subcore_barrier()

    for b in range(NBUF - 1):
      pltpu.async_copy(src_hbm.at[wid, b], sbuf[b], semsi[b])
      pltpu.async_copy(dst_hbm.at[wid, b], dbuf[b], semdi[b])
    for b in range(NBUF - 2):
      pltpu.make_async_copy(src_hbm.at[0, 0], sbuf[b], semsi[b]).wait()
      pltpu.async_copy(x_hbm.at[sbuf[b]], rows[b], semg[b])

    def body(j, carry):
      for b in range(NBUF):
        @pl.when(j % NBUF == b)
        def _():
          br = (b + NBUF - 1) % NBUF
          bg = (b + NBUF - 2) % NBUF

          @pl.when(j + NBUF - 1 < n_chunks)
          def _():
            @pl.when(j >= 1)
            def _():
              pltpu.make_async_copy(
                  rows[br], agg_sh.at[dbuf[br]], sems[br]).wait()
            pltpu.async_copy(src_hbm.at[wid, j + NBUF - 1], sbuf[br],
                             semsi[br])
            pltpu.async_copy(dst_hbm.at[wid, j + NBUF - 1], dbuf[br],
                             semdi[br])

          @pl.when(j + NBUF - 2 < n_chunks)
          def _():
            pltpu.make_async_copy(
                src_hbm.at[0, 0], sbuf[bg], semsi[bg]).wait()
            pltpu.async_copy(x_hbm.at[sbuf[bg]], rows[bg], semg[bg])

          pltpu.make_async_copy(x_hbm.at[pl.ds(0, CH)], rows[b],
                                semg[b]).wait()
          pltpu.make_async_copy(dst_hbm.at[0, 0], dbuf[b], semdi[b]).wait()
          pltpu.async_copy(rows[b], agg_sh.at[dbuf[b]], sems[b], add=True)
      return carry

    lax.fori_loop(0, n_chunks, body, 0)
    for b in range(NBUF):
      pltpu.make_async_copy(rows[b], agg_sh.at[dbuf[b]], sems[b]).wait()
    plsc.subcore_barrier()

    pltpu.sync_copy(agg_sh.at[pl.ds(r0, rows_per_tile)],
                    out_hbm.at[c, pl.ds(r0, rows_per_tile)])

  return sc_agg


def _mlp1_body(x_ref, p_ref, wa_ref, ba_ref, wb_ref, bb_ref, o_ref):
  h = x_ref[...] + p_ref[0] + p_ref[1]
  h = jnp.dot(h, wa_ref[...], preferred_element_type=jnp.float32)
  h = jnp.maximum(h + ba_ref[...], 0.0)
  o = jnp.dot(h, wb_ref[...], preferred_element_type=jnp.float32)
  o_ref[...] = jnp.maximum(o + bb_ref[...], 0.0)


def _mlp2_body(x_ref, p_ref, wa_ref, ba_ref, wb_ref, bb_ref, o_ref):
  h = x_ref[...] + p_ref[0] + p_ref[1]
  h = jnp.dot(h, wa_ref[...], preferred_element_type=jnp.float32)
  h = jnp.maximum(h + ba_ref[...], 0.0)
  o = jnp.dot(h, wb_ref[...], preferred_element_type=jnp.float32)
  o = o + bb_ref[...]
  m = jnp.max(o, axis=-1, keepdims=True)
  e = jnp.exp(o - m)
  o_ref[...] = e / jnp.sum(e, axis=-1, keepdims=True)


def _make_mlp(body, n, d_in, d_hid, d_out, blk):
  grid = n // blk
  assert n % blk == 0
  return pl.pallas_call(
      body,
      grid=(grid,),
      in_specs=[
          pl.BlockSpec((blk, d_in), lambda i: (i, 0)),
          pl.BlockSpec((NC, blk, d_in), lambda i: (0, i, 0)),
          pl.BlockSpec((d_in, d_hid), lambda i: (0, 0)),
          pl.BlockSpec((1, d_hid), lambda i: (0, 0)),
          pl.BlockSpec((d_hid, d_out), lambda i: (0, 0)),
          pl.BlockSpec((1, d_out), lambda i: (0, 0)),
      ],
      out_specs=pl.BlockSpec((blk, d_out), lambda i: (i, 0)),
      out_shape=jax.ShapeDtypeStruct((n, d_out), jnp.float32),
  )


def kernel(x, edge_index, W1a, b1a, W1b, b1b, W2a, b2a, W2b, b2b):
  n, d_in = x.shape
  d_hid = W1a.shape[1]
  d_out = W2b.shape[1]
  n_edges = edge_index.shape[1]

  epw = n_edges // NW
  src = edge_index[0].astype(jnp.int32)
  dst = edge_index[1].astype(jnp.int32)
  ch1, nb1 = 40, 5
  ch2, nb2 = 80, 5
  src1 = src.reshape(NW, epw // ch1, ch1)
  dst1 = dst.reshape(NW, epw // ch1, ch1)
  src2 = src.reshape(NW, epw // ch2, ch2)
  dst2 = dst.reshape(NW, epw // ch2, ch2)

  rpt = (n + NS * 8 - 1) // (NS * 8) * 8
  zeros1 = jnp.zeros((rpt, d_in), jnp.float32)
  zeros2 = jnp.zeros((rpt, d_hid), jnp.float32)

  agg1 = _make_sc_agg_ring(n, d_in, n_edges, ch2, 4, True)(x, src2, dst2, zeros1)
  h1 = _make_mlp(_mlp1_body, n, d_in, d_hid, d_hid, 2000)(
      x, agg1, W1a, b1a.reshape(1, -1), W1b, b1b.reshape(1, -1))
  agg2 = _make_sc_agg(n, d_hid, n_edges, ch2, nb2, False)(h1, src2, dst2, zeros2)
  out = _make_mlp(_mlp2_body, n, d_hid, d_hid, d_out, 2000)(
      h1, agg2, W2a, b2a.reshape(1, -1), W2b, b2b.reshape(1, -1))
  return out


# speedup vs baseline: 1.0724x; 1.0724x over previous
import functools

import jax
import jax.numpy as jnp
from jax import lax
from jax.experimental import pallas as pl
from jax.experimental.pallas import tpu as pltpu
from jax.experimental.pallas import tpu_sc as plsc

NC = 2
NS = 16
NW = NC * NS


def _make_sc_agg(n_nodes, d, n_edges, CH, NBUF, tc_tiling):
  epw = n_edges // NW
  n_chunks = epw // CH
  assert epw % CH == 0 and n_edges % NW == 0 and n_chunks % NBUF == 0
  rows_per_tile = (n_nodes + NS * 8 - 1) // (NS * 8) * 8
  n_pad = rows_per_tile * NS

  mesh = plsc.VectorSubcoreMesh(
      core_axis_name="c", subcore_axis_name="s", num_cores=NC,
      num_subcores=NS)

  @functools.partial(
      pl.kernel,
      out_type=jax.ShapeDtypeStruct((NC, n_pad, d), jnp.float32),
      mesh=mesh,
      compiler_params=pltpu.CompilerParams(use_tc_tiling_on_sc=tc_tiling),
      scratch_types=[
          pltpu.VMEM((n_chunks, CH), jnp.int32),
          pltpu.VMEM((n_chunks, CH), jnp.int32),
          *[pltpu.VMEM((CH, d), jnp.float32) for _ in range(NBUF)],
          pltpu.VMEM_SHARED((n_pad, d), jnp.float32),
          *[pltpu.SemaphoreType.DMA for _ in range(2 * NBUF)],
      ],
  )
  def sc_agg(x_hbm, ei_hbm, zeros_hbm, out_hbm,
             sidx, didx, *rest):
    rows = rest[:NBUF]
    agg_sh = rest[NBUF]
    semg = rest[NBUF + 1:2 * NBUF + 1]
    sems = rest[2 * NBUF + 1:]
    c = lax.axis_index("c")
    s = lax.axis_index("s")
    wid = s * NC + c
    r0 = s * rows_per_tile

    pltpu.sync_copy(ei_hbm.at[0, wid], sidx)
    pltpu.sync_copy(ei_hbm.at[1, wid], didx)
    pltpu.sync_copy(zeros_hbm, agg_sh.at[pl.ds(r0, rows_per_tile)])
    plsc.subcore_barrier()

    for b in range(NBUF - 1):
      pltpu.async_copy(x_hbm.at[sidx.at[b]], rows[b], semg[b])

    def outer(t, carry):
      for b in range(NBUF):
        j = t * NBUF + b
        bg = (b + NBUF - 1) % NBUF
        jj = j + NBUF - 1

        @pl.when(jj < n_chunks)
        def _():
          @pl.when(j >= 1)
          def _():
            pltpu.make_async_copy(
                rows[bg], agg_sh.at[didx.at[0]], sems[bg]).wait()
          pltpu.async_copy(x_hbm.at[sidx.at[jj]], rows[bg], semg[bg])

        pltpu.make_async_copy(
            x_hbm.at[pl.ds(0, CH)], rows[b], semg[b]).wait()
        pltpu.async_copy(rows[b], agg_sh.at[didx.at[j]], sems[b], add=True)
      return carry

    lax.fori_loop(0, n_chunks // NBUF, outer, 0)
    for b in range(NBUF):
      pltpu.make_async_copy(rows[b], agg_sh.at[didx.at[0]], sems[b]).wait()
    plsc.subcore_barrier()

    pltpu.sync_copy(agg_sh.at[pl.ds(r0, rows_per_tile)],
                    out_hbm.at[c, pl.ds(r0, rows_per_tile)])

  return sc_agg


def _make_sc_agg_ring(n_nodes, d, n_edges, CH, NBUF, tc_tiling):
  epw = n_edges // NW
  n_chunks = epw // CH
  assert epw % CH == 0 and n_edges % NW == 0 and n_chunks >= NBUF >= 3
  rows_per_tile = (n_nodes + NS * 8 - 1) // (NS * 8) * 8
  n_pad = rows_per_tile * NS

  mesh = plsc.VectorSubcoreMesh(
      core_axis_name="c", subcore_axis_name="s", num_cores=NC,
      num_subcores=NS)

  @functools.partial(
      pl.kernel,
      out_type=jax.ShapeDtypeStruct((NC, n_pad, d), jnp.float32),
      mesh=mesh,
      compiler_params=pltpu.CompilerParams(use_tc_tiling_on_sc=tc_tiling),
      scratch_types=[
          *[pltpu.VMEM((CH,), jnp.int32) for _ in range(NBUF)],
          *[pltpu.VMEM((CH,), jnp.int32) for _ in range(NBUF)],
          *[pltpu.VMEM((CH, d), jnp.float32) for _ in range(NBUF)],
          pltpu.VMEM_SHARED((n_pad, d), jnp.float32),
          *[pltpu.SemaphoreType.DMA for _ in range(4 * NBUF)],
      ],
  )
  def sc_agg(x_hbm, src_hbm, dst_hbm, zeros_hbm, out_hbm, *rest):
    sbuf = rest[:NBUF]
    dbuf = rest[NBUF:2 * NBUF]
    rows = rest[2 * NBUF:3 * NBUF]
    agg_sh = rest[3 * NBUF]
    semsi = rest[3 * NBUF + 1:4 * NBUF + 1]
    semdi = rest[4 * NBUF + 1:5 * NBUF + 1]
    semg = rest[5 * NBUF + 1:6 * NBUF + 1]
    sems = rest[6 * NBUF + 1:]
    c = lax.axis_index("c")
    s = lax.axis_index("s")
    wid = s * NC + c
    r0 = s * rows_per_tile

    pltpu.sync_copy(zeros_hbm, agg_sh.at[pl.ds(r0, rows_per_tile)])
    plsc.subcore_barrier()

    for b in range(NBUF - 1):
      pltpu.async_copy(src_hbm.at[wid, b], sbuf[b], semsi[b])
      pltpu.async_copy(dst_hbm.at[wid, b], dbuf[b], semdi[b])
    for b in range(NBUF - 2):
      pltpu.make_async_copy(src_hbm.at[0, 0], sbuf[b], semsi[b]).wait()
      pltpu.async_copy(x_hbm.at[sbuf[b]], rows[b], semg[b])

    def body(j, carry):
      for b in range(NBUF):
        @pl.when(j % NBUF == b)
        def _():
          br = (b + NBUF - 1) % NBUF
          bg = (b + NBUF - 2) % NBUF

          @pl.when(j + NBUF - 1 < n_chunks)
          def _():
            @pl.when(j >= 1)
            def _():
              pltpu.make_async_copy(
                  rows[br], agg_sh.at[dbuf[br]], sems[br]).wait()
            pltpu.async_copy(src_hbm.at[wid, j + NBUF - 1], sbuf[br],
                             semsi[br])
            pltpu.async_copy(dst_hbm.at[wid, j + NBUF - 1], dbuf[br],
                             semdi[br])

          @pl.when(j + NBUF - 2 < n_chunks)
          def _():
            pltpu.make_async_copy(
                src_hbm.at[0, 0], sbuf[bg], semsi[bg]).wait()
            pltpu.async_copy(x_hbm.at[sbuf[bg]], rows[bg], semg[bg])

          pltpu.make_async_copy(x_hbm.at[pl.ds(0, CH)], rows[b],
                                semg[b]).wait()
          pltpu.make_async_copy(dst_hbm.at[0, 0], dbuf[b], semdi[b]).wait()
          pltpu.async_copy(rows[b], agg_sh.at[dbuf[b]], sems[b], add=True)
      return carry

    lax.fori_loop(0, n_chunks, body, 0)
    for b in range(NBUF):
      pltpu.make_async_copy(rows[b], agg_sh.at[dbuf[b]], sems[b]).wait()
    plsc.subcore_barrier()

    pltpu.sync_copy(agg_sh.at[pl.ds(r0, rows_per_tile)],
                    out_hbm.at[c, pl.ds(r0, rows_per_tile)])

  return sc_agg


def _mlp1_body(x_ref, p_ref, wa_ref, ba_ref, wb_ref, bb_ref, o_ref):
  h = x_ref[...] + p_ref[0] + p_ref[1]
  h = jnp.dot(h, wa_ref[...], preferred_element_type=jnp.float32)
  h = jnp.maximum(h + ba_ref[...], 0.0)
  o = jnp.dot(h, wb_ref[...], preferred_element_type=jnp.float32)
  o_ref[...] = jnp.maximum(o + bb_ref[...], 0.0)


def _mlp2_body(x_ref, p_ref, wa_ref, ba_ref, wb_ref, bb_ref, o_ref):
  h = x_ref[...] + p_ref[0] + p_ref[1]
  h = jnp.dot(h, wa_ref[...], preferred_element_type=jnp.float32)
  h = jnp.maximum(h + ba_ref[...], 0.0)
  o = jnp.dot(h, wb_ref[...], preferred_element_type=jnp.float32)
  o = o + bb_ref[...]
  m = jnp.max(o, axis=-1, keepdims=True)
  e = jnp.exp(o - m)
  o_ref[...] = e / jnp.sum(e, axis=-1, keepdims=True)


def _make_mlp(body, n, d_in, d_hid, d_out, blk):
  grid = n // blk
  assert n % blk == 0
  return pl.pallas_call(
      body,
      grid=(grid,),
      in_specs=[
          pl.BlockSpec((blk, d_in), lambda i: (i, 0)),
          pl.BlockSpec((NC, blk, d_in), lambda i: (0, i, 0)),
          pl.BlockSpec((d_in, d_hid), lambda i: (0, 0)),
          pl.BlockSpec((1, d_hid), lambda i: (0, 0)),
          pl.BlockSpec((d_hid, d_out), lambda i: (0, 0)),
          pl.BlockSpec((1, d_out), lambda i: (0, 0)),
      ],
      out_specs=pl.BlockSpec((blk, d_out), lambda i: (i, 0)),
      out_shape=jax.ShapeDtypeStruct((n, d_out), jnp.float32),
  )


def kernel(x, edge_index, W1a, b1a, W1b, b1b, W2a, b2a, W2b, b2b):
  n, d_in = x.shape
  d_hid = W1a.shape[1]
  d_out = W2b.shape[1]
  n_edges = edge_index.shape[1]

  epw = n_edges // NW
  ei = edge_index.astype(jnp.int32)
  ch1, nb1 = 40, 5
  ch2, nb2 = 80, 5
  ei1 = ei.reshape(2, NW, epw // ch1, ch1)
  ei2 = ei.reshape(2, NW, epw // ch2, ch2)

  rpt = (n + NS * 8 - 1) // (NS * 8) * 8
  zeros1 = jnp.zeros((rpt, d_in), jnp.float32)
  zeros2 = jnp.zeros((rpt, d_hid), jnp.float32)

  agg1 = _make_sc_agg(n, d_in, n_edges, ch1, nb1, False)(x, ei1, zeros1)
  h1 = _make_mlp(_mlp1_body, n, d_in, d_hid, d_hid, 2000)(
      x, agg1, W1a, b1a.reshape(1, -1), W1b, b1b.reshape(1, -1))
  agg2 = _make_sc_agg(n, d_hid, n_edges, ch2, nb2, False)(h1, ei2, zeros2)
  out = _make_mlp(_mlp2_body, n, d_hid, d_hid, d_out, 2000)(
      h1, agg2, W2a, b2a.reshape(1, -1), W2b, b2b.reshape(1, -1))
  return out
